# baseline (device time: 7981 ns/iter reference)
import jax
import jax.numpy as jnp
from jax import lax
from jax.experimental import pallas as pl
from jax.experimental.pallas import tpu as pltpu

N_DEV = 4
EPS = 1e-5
N_GLOBAL = 1024.0


def kernel(x, gamma):
    m, n_per = x.shape

    def body(x_hbm, g_hbm, out_hbm, x_vmem, g_vmem, out_vmem,
             comm_ref, local_sems, send_sems, recv_sems):
        my_pos = lax.axis_index("i")

        x_copy = pltpu.make_async_copy(x_hbm, x_vmem, local_sems.at[0])
        g_copy = pltpu.make_async_copy(g_hbm, g_vmem, local_sems.at[1])
        x_copy.start()
        g_copy.start()

        barrier_sem = pltpu.get_barrier_semaphore()
        for k in range(1, N_DEV):
            peer = lax.rem(my_pos + k, N_DEV)
            pl.semaphore_signal(
                barrier_sem, inc=1, device_id=(peer,),
                device_id_type=pl.DeviceIdType.MESH,
            )
        pl.semaphore_wait(barrier_sem, N_DEV - 1)

        x_copy.wait()
        xf = x_vmem[:, :]
        p = jnp.sum(xf * xf, axis=1).reshape(1, m)
        comm_ref[my_pos] = p

        sends = []
        for k in range(1, N_DEV):
            peer = lax.rem(my_pos + k, N_DEV)
            rdma = pltpu.make_async_remote_copy(
                src_ref=comm_ref.at[my_pos],
                dst_ref=comm_ref.at[my_pos],
                send_sem=send_sems.at[k - 1],
                recv_sem=recv_sems.at[k - 1],
                device_id=(peer,),
                device_id_type=pl.DeviceIdType.MESH,
            )
            rdma.start()
            sends.append(rdma)

        g_copy.wait()
        xg = xf * g_vmem[:][None, :]

        for k in range(1, N_DEV):
            src_pos = lax.rem(my_pos - k + N_DEV, N_DEV)
            recv = pltpu.make_async_remote_copy(
                src_ref=comm_ref.at[src_pos],
                dst_ref=comm_ref.at[src_pos],
                send_sem=send_sems.at[k - 1],
                recv_sem=recv_sems.at[k - 1],
                device_id=(src_pos,),
                device_id_type=pl.DeviceIdType.MESH,
            )
            recv.wait_recv()

        total = comm_ref[0] + comm_ref[1] + comm_ref[2] + comm_ref[3]
        inv = lax.rsqrt(total * (1.0 / N_GLOBAL) + EPS)
        out_vmem[:, :] = (xg * inv.reshape(m, 1)).astype(jnp.bfloat16)

        out_copy = pltpu.make_async_copy(out_vmem, out_hbm, local_sems.at[2])
        out_copy.start()
        for rdma in sends:
            rdma.wait_send()
        out_copy.wait()

    return pl.pallas_call(
        body,
        out_shape=jax.ShapeDtypeStruct((m, n_per), jnp.bfloat16),
        in_specs=[
            pl.BlockSpec(memory_space=pl.ANY),
            pl.BlockSpec(memory_space=pl.ANY),
        ],
        out_specs=pl.BlockSpec(memory_space=pl.ANY),
        scratch_shapes=[
            pltpu.VMEM((m, n_per), jnp.float32),
            pltpu.VMEM((n_per,), jnp.float32),
            pltpu.VMEM((m, n_per), jnp.bfloat16),
            pltpu.VMEM((N_DEV, 1, m), jnp.float32),
            pltpu.SemaphoreType.DMA((3,)),
            pltpu.SemaphoreType.DMA((N_DEV - 1,)),
            pltpu.SemaphoreType.DMA((N_DEV - 1,)),
        ],
        compiler_params=pltpu.CompilerParams(collective_id=0),
    )(x, gamma)


# device time: 6908 ns/iter; 1.1553x vs baseline; 1.1553x over previous
import jax
import jax.numpy as jnp
from jax import lax
from jax.experimental import pallas as pl
from jax.experimental.pallas import tpu as pltpu

N_DEV = 4
EPS = 1e-5
N_GLOBAL = 1024.0


def kernel(x, gamma):
    m, n_per = x.shape
    x = pltpu.with_memory_space_constraint(x, pltpu.MemorySpace.HBM)
    gamma = pltpu.with_memory_space_constraint(gamma, pltpu.MemorySpace.HBM)

    def body(x_hbm, g_hbm, out_hbm, x_vmem, g_vmem, out_vmem,
             comm_ref, local_sems, send_sems, recv_sems):
        my_pos = lax.axis_index("i")

        x_copy = pltpu.make_async_copy(x_hbm, x_vmem, local_sems.at[0])
        g_copy = pltpu.make_async_copy(g_hbm, g_vmem, local_sems.at[1])
        x_copy.start()
        g_copy.start()

        barrier_sem = pltpu.get_barrier_semaphore()
        for k in range(1, N_DEV):
            peer = lax.rem(my_pos + k, N_DEV)
            pl.semaphore_signal(
                barrier_sem, inc=1, device_id=(peer,),
                device_id_type=pl.DeviceIdType.MESH,
            )
        pl.semaphore_wait(barrier_sem, N_DEV - 1)

        x_copy.wait()
        xf = x_vmem[:, :]
        p = jnp.sum(xf * xf, axis=1).reshape(1, m)
        comm_ref[my_pos] = p

        sends = []
        for k in range(1, N_DEV):
            peer = lax.rem(my_pos + k, N_DEV)
            rdma = pltpu.make_async_remote_copy(
                src_ref=comm_ref.at[my_pos],
                dst_ref=comm_ref.at[my_pos],
                send_sem=send_sems.at[k - 1],
                recv_sem=recv_sems.at[k - 1],
                device_id=(peer,),
                device_id_type=pl.DeviceIdType.MESH,
            )
            rdma.start()
            sends.append(rdma)

        g_copy.wait()
        xg = xf * g_vmem[:][None, :]

        for k in range(1, N_DEV):
            src_pos = lax.rem(my_pos - k + N_DEV, N_DEV)
            recv = pltpu.make_async_remote_copy(
                src_ref=comm_ref.at[src_pos],
                dst_ref=comm_ref.at[src_pos],
                send_sem=send_sems.at[k - 1],
                recv_sem=recv_sems.at[k - 1],
                device_id=(src_pos,),
                device_id_type=pl.DeviceIdType.MESH,
            )
            recv.wait_recv()

        total = comm_ref[0] + comm_ref[1] + comm_ref[2] + comm_ref[3]
        inv = lax.rsqrt(total * (1.0 / N_GLOBAL) + EPS)
        out_vmem[:, :] = (xg * inv.reshape(m, 1)).astype(jnp.bfloat16)

        out_copy = pltpu.make_async_copy(out_vmem, out_hbm, local_sems.at[2])
        out_copy.start()
        for rdma in sends:
            rdma.wait_send()
        out_copy.wait()

    return pl.pallas_call(
        body,
        out_shape=jax.ShapeDtypeStruct((m, n_per), jnp.bfloat16),
        in_specs=[
            pl.BlockSpec(memory_space=pltpu.MemorySpace.HBM),
            pl.BlockSpec(memory_space=pltpu.MemorySpace.HBM),
        ],
        out_specs=pl.BlockSpec(memory_space=pltpu.MemorySpace.HBM),
        scratch_shapes=[
            pltpu.VMEM((m, n_per), jnp.float32),
            pltpu.VMEM((n_per,), jnp.float32),
            pltpu.VMEM((m, n_per), jnp.bfloat16),
            pltpu.VMEM((N_DEV, 1, m), jnp.float32),
            pltpu.SemaphoreType.DMA((3,)),
            pltpu.SemaphoreType.DMA((N_DEV - 1,)),
            pltpu.SemaphoreType.DMA((N_DEV - 1,)),
        ],
        compiler_params=pltpu.CompilerParams(collective_id=0),
    )(x, gamma)


# device time: 6889 ns/iter; 1.1585x vs baseline; 1.0028x over previous
import jax
import jax.numpy as jnp
from jax import lax
from jax.experimental import pallas as pl
from jax.experimental.pallas import tpu as pltpu

N_DEV = 4
EPS = 1e-5
N_GLOBAL = 1024.0


def kernel(x, gamma):
    m, n_per = x.shape
    x = pltpu.with_memory_space_constraint(x, pltpu.MemorySpace.HBM)
    gamma = pltpu.with_memory_space_constraint(gamma, pltpu.MemorySpace.HBM)

    def body(x_hbm, g_hbm, out_hbm, x_vmem, g_vmem, out_vmem,
             comm_ref, local_sems, send_sems, recv_sems):
        my_pos = lax.axis_index("i")

        x_copy = pltpu.make_async_copy(x_hbm, x_vmem, local_sems.at[0])
        g_copy = pltpu.make_async_copy(g_hbm, g_vmem, local_sems.at[1])
        x_copy.start()
        g_copy.start()

        barrier_sem = pltpu.get_barrier_semaphore()
        for k in range(1, N_DEV):
            peer = lax.rem(my_pos + k, N_DEV)
            pl.semaphore_signal(
                barrier_sem, inc=1, device_id=(peer,),
                device_id_type=pl.DeviceIdType.MESH,
            )

        x_copy.wait()
        xf = x_vmem[:, :]
        p = jnp.sum(xf * xf, axis=1).reshape(1, m)
        comm_ref[my_pos] = p

        pl.semaphore_wait(barrier_sem, N_DEV - 1)

        sends = []
        for k in range(1, N_DEV):
            peer = lax.rem(my_pos + k, N_DEV)
            rdma = pltpu.make_async_remote_copy(
                src_ref=comm_ref.at[my_pos],
                dst_ref=comm_ref.at[my_pos],
                send_sem=send_sems.at[k - 1],
                recv_sem=recv_sems.at[k - 1],
                device_id=(peer,),
                device_id_type=pl.DeviceIdType.MESH,
            )
            rdma.start()
            sends.append(rdma)

        g_copy.wait()
        xg = xf * g_vmem[:][None, :]

        for k in range(1, N_DEV):
            src_pos = lax.rem(my_pos - k + N_DEV, N_DEV)
            recv = pltpu.make_async_remote_copy(
                src_ref=comm_ref.at[src_pos],
                dst_ref=comm_ref.at[src_pos],
                send_sem=send_sems.at[k - 1],
                recv_sem=recv_sems.at[k - 1],
                device_id=(src_pos,),
                device_id_type=pl.DeviceIdType.MESH,
            )
            recv.wait_recv()

        total = comm_ref[0] + comm_ref[1] + comm_ref[2] + comm_ref[3]
        inv = lax.rsqrt(total * (1.0 / N_GLOBAL) + EPS)
        out_vmem[:, :] = (xg * inv.reshape(m, 1)).astype(jnp.bfloat16)

        out_copy = pltpu.make_async_copy(out_vmem, out_hbm, local_sems.at[2])
        out_copy.start()
        for rdma in sends:
            rdma.wait_send()
        out_copy.wait()

    return pl.pallas_call(
        body,
        out_shape=jax.ShapeDtypeStruct((m, n_per), jnp.bfloat16),
        in_specs=[
            pl.BlockSpec(memory_space=pltpu.MemorySpace.HBM),
            pl.BlockSpec(memory_space=pltpu.MemorySpace.HBM),
        ],
        out_specs=pl.BlockSpec(memory_space=pl.ANY),
        scratch_shapes=[
            pltpu.VMEM((m, n_per), jnp.float32),
            pltpu.VMEM((n_per,), jnp.float32),
            pltpu.VMEM((m, n_per), jnp.bfloat16),
            pltpu.VMEM((N_DEV, 1, m), jnp.float32),
            pltpu.SemaphoreType.DMA((3,)),
            pltpu.SemaphoreType.DMA((N_DEV - 1,)),
            pltpu.SemaphoreType.DMA((N_DEV - 1,)),
        ],
        compiler_params=pltpu.CompilerParams(collective_id=0),
    )(x, gamma)


# device time: 6881 ns/iter; 1.1599x vs baseline; 1.0012x over previous
import jax
import jax.numpy as jnp
from jax import lax
from jax.experimental import pallas as pl
from jax.experimental.pallas import tpu as pltpu

N_DEV = 4
EPS = 1e-5
N_GLOBAL = 1024.0


def kernel(x, gamma):
    m, n_per = x.shape
    x = pltpu.with_memory_space_constraint(x, pltpu.MemorySpace.HBM)
    gamma = pltpu.with_memory_space_constraint(gamma, pltpu.MemorySpace.HBM)

    def body(x_hbm, g_hbm, out_ref, x_vmem, g_vmem,
             comm_ref, local_sems, send_sems, recv_sems):
        my_pos = lax.axis_index("i")

        x_copy = pltpu.make_async_copy(x_hbm, x_vmem, local_sems.at[0])
        g_copy = pltpu.make_async_copy(g_hbm, g_vmem, local_sems.at[1])
        x_copy.start()
        g_copy.start()

        barrier_sem = pltpu.get_barrier_semaphore()
        for k in range(1, N_DEV):
            peer = lax.rem(my_pos + k, N_DEV)
            pl.semaphore_signal(
                barrier_sem, inc=1, device_id=(peer,),
                device_id_type=pl.DeviceIdType.MESH,
            )

        x_copy.wait()
        xf = x_vmem[:, :]
        p = jnp.sum(xf * xf, axis=1).reshape(1, m)
        comm_ref[my_pos] = p

        pl.semaphore_wait(barrier_sem, N_DEV - 1)

        sends = []
        for k in range(1, N_DEV):
            peer = lax.rem(my_pos + k, N_DEV)
            rdma = pltpu.make_async_remote_copy(
                src_ref=comm_ref.at[my_pos],
                dst_ref=comm_ref.at[my_pos],
                send_sem=send_sems.at[k - 1],
                recv_sem=recv_sems.at[k - 1],
                device_id=(peer,),
                device_id_type=pl.DeviceIdType.MESH,
            )
            rdma.start()
            sends.append(rdma)

        g_copy.wait()
        xg = xf * g_vmem[:][None, :]

        for k in range(1, N_DEV):
            src_pos = lax.rem(my_pos - k + N_DEV, N_DEV)
            recv = pltpu.make_async_remote_copy(
                src_ref=comm_ref.at[src_pos],
                dst_ref=comm_ref.at[src_pos],
                send_sem=send_sems.at[k - 1],
                recv_sem=recv_sems.at[k - 1],
                device_id=(src_pos,),
                device_id_type=pl.DeviceIdType.MESH,
            )
            recv.wait_recv()

        total = comm_ref[0] + comm_ref[1] + comm_ref[2] + comm_ref[3]
        inv = lax.rsqrt(total * (1.0 / N_GLOBAL) + EPS)
        out_ref[:, :] = (xg * inv.reshape(m, 1)).astype(jnp.bfloat16)

        for rdma in sends:
            rdma.wait_send()

    return pl.pallas_call(
        body,
        out_shape=jax.ShapeDtypeStruct((m, n_per), jnp.bfloat16),
        in_specs=[
            pl.BlockSpec(memory_space=pltpu.MemorySpace.HBM),
            pl.BlockSpec(memory_space=pltpu.MemorySpace.HBM),
        ],
        out_specs=pl.BlockSpec(memory_space=pltpu.VMEM),
        scratch_shapes=[
            pltpu.VMEM((m, n_per), jnp.float32),
            pltpu.VMEM((n_per,), jnp.float32),
            pltpu.VMEM((N_DEV, 1, m), jnp.float32),
            pltpu.SemaphoreType.DMA((2,)),
            pltpu.SemaphoreType.DMA((N_DEV - 1,)),
            pltpu.SemaphoreType.DMA((N_DEV - 1,)),
        ],
        compiler_params=pltpu.CompilerParams(collective_id=0),
    )(x, gamma)


# device time: 6857 ns/iter; 1.1639x vs baseline; 1.0035x over previous
import jax
import jax.numpy as jnp
from jax import lax
from jax.experimental import pallas as pl
from jax.experimental.pallas import tpu as pltpu

N_DEV = 4
EPS = 1e-5
N_GLOBAL = 1024.0


def kernel(x, gamma):
    m, n_per = x.shape
    x = pltpu.with_memory_space_constraint(x, pltpu.MemorySpace.HBM)
    gamma = pltpu.with_memory_space_constraint(gamma, pltpu.MemorySpace.HBM)

    def body(x_hbm, g_hbm, out_ref, x_vmem, g_vmem,
             comm_ref, local_sems, send_sems, recv_sems, barrier_sem):
        my_pos = lax.axis_index("i")

        x_copy = pltpu.make_async_copy(x_hbm, x_vmem, local_sems.at[0])
        g_copy = pltpu.make_async_copy(g_hbm, g_vmem, local_sems.at[1])
        x_copy.start()
        g_copy.start()

        for k in range(1, N_DEV):
            peer = lax.rem(my_pos + k, N_DEV)
            pl.semaphore_signal(
                barrier_sem, inc=1, device_id=(peer,),
                device_id_type=pl.DeviceIdType.MESH,
            )

        x_copy.wait()
        xf = x_vmem[:, :]
        p = jnp.sum(xf * xf, axis=1).reshape(1, m)
        comm_ref[my_pos] = p

        pl.semaphore_wait(barrier_sem, N_DEV - 1)

        sends = []
        for k in range(1, N_DEV):
            peer = lax.rem(my_pos + k, N_DEV)
            rdma = pltpu.make_async_remote_copy(
                src_ref=comm_ref.at[my_pos],
                dst_ref=comm_ref.at[my_pos],
                send_sem=send_sems.at[k - 1],
                recv_sem=recv_sems.at[k - 1],
                device_id=(peer,),
                device_id_type=pl.DeviceIdType.MESH,
            )
            rdma.start()
            sends.append(rdma)

        g_copy.wait()
        xg = xf * g_vmem[:][None, :]

        for k in range(1, N_DEV):
            src_pos = lax.rem(my_pos - k + N_DEV, N_DEV)
            recv = pltpu.make_async_remote_copy(
                src_ref=comm_ref.at[src_pos],
                dst_ref=comm_ref.at[src_pos],
                send_sem=send_sems.at[k - 1],
                recv_sem=recv_sems.at[k - 1],
                device_id=(src_pos,),
                device_id_type=pl.DeviceIdType.MESH,
            )
            recv.wait_recv()

        total = comm_ref[0] + comm_ref[1] + comm_ref[2] + comm_ref[3]
        inv = lax.rsqrt(total * (1.0 / N_GLOBAL) + EPS)
        out_ref[:, :] = (xg * inv.reshape(m, 1)).astype(jnp.bfloat16)

        for rdma in sends:
            rdma.wait_send()

    return pl.pallas_call(
        body,
        out_shape=jax.ShapeDtypeStruct((m, n_per), jnp.bfloat16),
        in_specs=[
            pl.BlockSpec(memory_space=pltpu.MemorySpace.HBM),
            pl.BlockSpec(memory_space=pltpu.MemorySpace.HBM),
        ],
        out_specs=pl.BlockSpec(memory_space=pltpu.VMEM),
        scratch_shapes=[
            pltpu.VMEM((m, n_per), jnp.float32),
            pltpu.VMEM((n_per,), jnp.float32),
            pltpu.VMEM((N_DEV, 1, m), jnp.float32),
            pltpu.SemaphoreType.DMA((2,)),
            pltpu.SemaphoreType.DMA((N_DEV - 1,)),
            pltpu.SemaphoreType.DMA((N_DEV - 1,)),
            pltpu.SemaphoreType.REGULAR,
        ],
        compiler_params=pltpu.CompilerParams(skip_device_barrier=True),
    )(x, gamma)


# device time: 6845 ns/iter; 1.1660x vs baseline; 1.0018x over previous
import jax
import jax.numpy as jnp
from jax import lax
from jax.experimental import pallas as pl
from jax.experimental.pallas import tpu as pltpu

N_DEV = 4
EPS = 1e-5
N_GLOBAL = 1024.0


def kernel(x, gamma):
    m, n_per = x.shape
    x = pltpu.with_memory_space_constraint(x, pltpu.MemorySpace.HBM)
    gamma = pltpu.with_memory_space_constraint(gamma, pltpu.MemorySpace.HBM)

    def body(x_hbm, g_hbm, out_ref, x_vmem, g_vmem,
             comm_ref, local_sems, send_sems, recv_sems):
        my_pos = lax.axis_index("i")

        h = m // 2
        x_copy_a = pltpu.make_async_copy(
            x_hbm.at[0:h], x_vmem.at[0:h], local_sems.at[0])
        x_copy_b = pltpu.make_async_copy(
            x_hbm.at[h:m], x_vmem.at[h:m], local_sems.at[2])
        g_copy = pltpu.make_async_copy(g_hbm, g_vmem, local_sems.at[1])
        x_copy_a.start()
        x_copy_b.start()
        g_copy.start()

        barrier_sem = pltpu.get_barrier_semaphore()
        for k in range(1, N_DEV):
            peer = lax.rem(my_pos + k, N_DEV)
            pl.semaphore_signal(
                barrier_sem, inc=1, device_id=(peer,),
                device_id_type=pl.DeviceIdType.MESH,
            )

        x_copy_a.wait()
        xa = x_vmem[0:h, :]
        pa = jnp.sum(xa * xa, axis=1).reshape(1, h)
        x_copy_b.wait()
        xb = x_vmem[h:m, :]
        pb = jnp.sum(xb * xb, axis=1).reshape(1, h)
        comm_ref[my_pos, :, 0:h] = pa
        comm_ref[my_pos, :, h:m] = pb

        pl.semaphore_wait(barrier_sem, N_DEV - 1)

        sends = []
        for k in (2, 1, 3):
            peer = lax.rem(my_pos + k, N_DEV)
            rdma = pltpu.make_async_remote_copy(
                src_ref=comm_ref.at[my_pos],
                dst_ref=comm_ref.at[my_pos],
                send_sem=send_sems.at[k - 1],
                recv_sem=recv_sems.at[k - 1],
                device_id=(peer,),
                device_id_type=pl.DeviceIdType.MESH,
            )
            rdma.start()
            sends.append(rdma)

        g_copy.wait()
        xg = x_vmem[:, :] * g_vmem[:][None, :]

        for k in range(1, N_DEV):
            src_pos = lax.rem(my_pos - k + N_DEV, N_DEV)
            recv = pltpu.make_async_remote_copy(
                src_ref=comm_ref.at[src_pos],
                dst_ref=comm_ref.at[src_pos],
                send_sem=send_sems.at[k - 1],
                recv_sem=recv_sems.at[k - 1],
                device_id=(src_pos,),
                device_id_type=pl.DeviceIdType.MESH,
            )
            recv.wait_recv()

        total = comm_ref[0] + comm_ref[1] + comm_ref[2] + comm_ref[3]
        inv = lax.rsqrt(total * (1.0 / N_GLOBAL) + EPS)
        out_ref[:, :] = (xg * inv.reshape(m, 1)).astype(jnp.bfloat16)

        for rdma in sends:
            rdma.wait_send()

    return pl.pallas_call(
        body,
        out_shape=jax.ShapeDtypeStruct((m, n_per), jnp.bfloat16),
        in_specs=[
            pl.BlockSpec(memory_space=pltpu.MemorySpace.HBM),
            pl.BlockSpec(memory_space=pltpu.MemorySpace.HBM),
        ],
        out_specs=pl.BlockSpec(memory_space=pltpu.VMEM),
        scratch_shapes=[
            pltpu.VMEM((m, n_per), jnp.float32),
            pltpu.VMEM((n_per,), jnp.float32),
            pltpu.VMEM((N_DEV, 1, m), jnp.float32),
            pltpu.SemaphoreType.DMA((3,)),
            pltpu.SemaphoreType.DMA((N_DEV - 1,)),
            pltpu.SemaphoreType.DMA((N_DEV - 1,)),
        ],
        compiler_params=pltpu.CompilerParams(collective_id=0),
    )(x, gamma)
